# baseline (speedup 1.0000x reference)
import jax
import jax.numpy as jnp
from jax.experimental import pallas as pl

B = 4096
V = 6890
NROWS = 37
BB = 1024


def _body(j37_ref, vt_ref, jt_ref, out_ref):
    plane = vt_ref[0]
    reg = jnp.dot(j37_ref[...], plane,
                  preferred_element_type=jnp.float32)
    out_ref[0, :24, :] = jt_ref[0]
    out_ref[0, 24:, :] = reg


def kernel(vertices, joints, extra_joints_idxs, J_regressor_extra9,
           J_regressor_h36m17):
    vt = jnp.transpose(vertices, (2, 1, 0))
    jt = jnp.transpose(joints, (2, 1, 0))

    onehot = jax.nn.one_hot(extra_joints_idxs, V, dtype=jnp.float32)
    j37 = jnp.concatenate([onehot, J_regressor_extra9, J_regressor_h36m17],
                          axis=0)

    out_t = pl.pallas_call(
        _body,
        grid=(3, B // BB),
        in_specs=[
            pl.BlockSpec((NROWS, V), lambda k, b: (0, 0)),
            pl.BlockSpec((1, V, BB), lambda k, b: (k, 0, b)),
            pl.BlockSpec((1, 24, BB), lambda k, b: (k, 0, b)),
        ],
        out_specs=pl.BlockSpec((1, 61, BB), lambda k, b: (k, 0, b)),
        out_shape=jax.ShapeDtypeStruct((3, 61, B), jnp.float32),
    )(j37, vt, jt)

    return jnp.transpose(out_t, (2, 1, 0))


# speedup vs baseline: 1.0701x; 1.0701x over previous
import jax
import jax.numpy as jnp
from jax.experimental import pallas as pl

B = 4096
V = 6890
NROWS = 37
BB = 256


def _body(j37_ref, vt_ref, jt_ref, out_ref):
    plane = vt_ref[0]
    reg = jnp.dot(j37_ref[...], plane,
                  preferred_element_type=jnp.float32)
    out_ref[0, :24, :] = jt_ref[0]
    out_ref[0, 24:, :] = reg


def kernel(vertices, joints, extra_joints_idxs, J_regressor_extra9,
           J_regressor_h36m17):
    vt = jnp.transpose(vertices, (2, 1, 0))
    jt = jnp.transpose(joints, (2, 1, 0))

    onehot = jax.nn.one_hot(extra_joints_idxs, V, dtype=jnp.float32)
    j37 = jnp.concatenate([onehot, J_regressor_extra9, J_regressor_h36m17],
                          axis=0)

    out_t = pl.pallas_call(
        _body,
        grid=(3, B // BB),
        in_specs=[
            pl.BlockSpec((NROWS, V), lambda k, b: (0, 0)),
            pl.BlockSpec((1, V, BB), lambda k, b: (k, 0, b)),
            pl.BlockSpec((1, 24, BB), lambda k, b: (k, 0, b)),
        ],
        out_specs=pl.BlockSpec((1, 61, BB), lambda k, b: (k, 0, b)),
        out_shape=jax.ShapeDtypeStruct((3, 61, B), jnp.float32),
    )(j37, vt, jt)

    return jnp.transpose(out_t, (2, 1, 0))
